# baseline (speedup 1.0000x reference)
import functools

import jax
import jax.numpy as jnp
import numpy as np
from jax import lax
from jax.experimental import pallas as pl
from jax.experimental.pallas import tpu as pltpu
from jax.experimental.pallas import tpu_sc as plsc

N = 10000
E = 320000
H = 128
NG = 64

NC = 2
NS = 16
K = 16
EPW = E // (NC * NS)
NCHUNK = EPW // K
NSLOT = 6
LDA = 2
IXA = 4
ZC = 16
NZCH = N // ZC

BE = 512
BN_ = 1000


def _conv_sc_body(h_hbm, eterm_hbm, src_hbm, dst_hbm, out_hbm, *rest):
    srcv = list(rest[0:NSLOT])
    dstv = list(rest[NSLOT:2 * NSLOT])
    ebufs = list(rest[2 * NSLOT:3 * NSLOT])
    hbufs = list(rest[3 * NSLOT:4 * NSLOT])
    aggr_sh = rest[4 * NSLOT]
    isems = list(rest[4 * NSLOT + 1:5 * NSLOT + 1])
    lsems = list(rest[5 * NSLOT + 1:6 * NSLOT + 1])
    ssems = list(rest[6 * NSLOT + 1:7 * NSLOT + 1])
    c = lax.axis_index("c")
    s = lax.axis_index("s")
    wid = s * NC + c
    zbuf = hbufs[0]

    def wait_rows(dst_ref, sem):
        pltpu.make_async_copy(h_hbm.at[pl.ds(0, K)], dst_ref, sem).wait()

    def wait_eterm(dst_ref, sem):
        pltpu.make_async_copy(eterm_hbm.at[pl.ds(0, K // 2)], dst_ref,
                              sem).wait()

    def wait_idx(dst_ref, sem):
        pltpu.make_async_copy(src_hbm.at[0], dst_ref, sem).wait()

    def zrow(r, carry):
        for v in range(H // 16):
            zbuf[r, pl.ds(v * 16, 16)] = jnp.zeros((16,), jnp.float32)
        return carry
    lax.fori_loop(0, ZC, zrow, 0)

    def zchunk(t, carry):
        j = s + t * NS

        @pl.when(j < NZCH)
        def _():
            pltpu.sync_copy(zbuf.at[pl.ds(0, ZC)],
                            aggr_sh.at[pl.ds(j * ZC, ZC)])
        return carry
    lax.fori_loop(0, (NZCH + NS - 1) // NS, zchunk, 0)
    plsc.subcore_barrier()

    pbase = wid * (EPW // 2)
    cbase = wid * NCHUNK

    def issue_src_idx(j, sl):
        pltpu.async_copy(src_hbm.at[cbase + j], srcv[sl], isems[sl])

    def issue_loads(j, sl):
        pltpu.async_copy(dst_hbm.at[cbase + j], dstv[sl], lsems[sl])
        pltpu.async_copy(eterm_hbm.at[pl.ds(pbase + j * (K // 2), K // 2)],
                         ebufs[sl], lsems[sl])
        pltpu.async_copy(h_hbm.at[srcv[sl]], hbufs[sl], lsems[sl])

    def relu_add(sl):
        def row(r, rc):
            for g in range(H // 16):
                vi = ebufs[sl][r, pl.ds(16 * g, 16)]
                ea = lax.bitcast_convert_type(jnp.left_shift(vi, 16),
                                              jnp.float32)
                eb = lax.bitcast_convert_type(vi & jnp.int32(-65536),
                                              jnp.float32)
                sg = pl.ds(16 * g, 16)
                hbufs[sl][2 * r, sg] = jnp.maximum(
                    hbufs[sl][2 * r, sg] + ea, 0.0)
                hbufs[sl][2 * r + 1, sg] = jnp.maximum(
                    hbufs[sl][2 * r + 1, sg] + eb, 0.0)
            return rc
        lax.fori_loop(0, K // 2, row, 0)

    def scatter(j, sl):
        pltpu.async_copy(hbufs[sl], aggr_sh.at[dstv[sl]], ssems[sl],
                         add=True)

    for j0 in range(LDA):
        pltpu.sync_copy(src_hbm.at[cbase + j0], srcv[j0])
    for j0 in range(LDA, IXA):
        issue_src_idx(j0, j0 % NSLOT)
    for j0 in range(LDA):
        issue_loads(j0, j0)

    def grp(t, carry):
        for u in range(NSLOT):
            j = NSLOT * t + u
            sp = (u + IXA) % NSLOT
            sn = (u + LDA) % NSLOT

            @pl.when(j + IXA < NCHUNK)
            def _():
                issue_src_idx(j + IXA, sp)

            @pl.when((j >= NSLOT - LDA) & (j + LDA < NCHUNK))
            def _():
                wait_rows(hbufs[sn], ssems[sn])

            @pl.when(j + LDA < NCHUNK)
            def _():
                wait_idx(srcv[sn], isems[sn])
                issue_loads(j + LDA, sn)

            wait_idx(dstv[u], lsems[u])
            wait_eterm(ebufs[u], lsems[u])
            wait_rows(hbufs[u], lsems[u])
            relu_add(u)
            scatter(j, u)
        return carry
    lax.fori_loop(0, NCHUNK // NSLOT, grp, 0)
    for j in range((NCHUNK // NSLOT) * NSLOT, NCHUNK):
        u = j % NSLOT
        wait_idx(dstv[u], lsems[u])
        wait_eterm(ebufs[u], lsems[u])
        wait_rows(hbufs[u], lsems[u])
        relu_add(u)
        scatter(j, u)
    for j in range(NCHUNK - NSLOT, NCHUNK):
        wait_rows(hbufs[j % NSLOT], ssems[j % NSLOT])
    plsc.subcore_barrier()

    def exp(t, carry):
        j = s + t * NS

        @pl.when(j < NZCH)
        def _():
            pltpu.sync_copy(aggr_sh.at[pl.ds(j * ZC, ZC)],
                            zbuf.at[pl.ds(0, ZC)])
            pltpu.sync_copy(zbuf.at[pl.ds(0, ZC)],
                            out_hbm.at[pl.ds(c * N + j * ZC, ZC)])
        return carry
    lax.fori_loop(0, (NZCH + NS - 1) // NS, exp, 0)


_conv_sc_cache = []


def _conv_sc(h, eterm, src, dst):
    if not _conv_sc_cache:
        _conv_sc_cache.append(functools.partial(
            pl.kernel,
            mesh=plsc.VectorSubcoreMesh(core_axis_name="c",
                                        subcore_axis_name="s"),
            out_type=jax.ShapeDtypeStruct((NC * N, H), jnp.float32),
            scratch_types=(
                [pltpu.VMEM((K,), jnp.int32)] * NSLOT
                + [pltpu.VMEM((K,), jnp.int32)] * NSLOT
                + [pltpu.VMEM((K // 2, H), jnp.int32)] * NSLOT
                + [pltpu.VMEM((K, H), jnp.float32)] * NSLOT
                + [pltpu.VMEM_SHARED((N, H), jnp.float32)]
                + [pltpu.SemaphoreType.DMA] * (3 * NSLOT)
            ),
        )(_conv_sc_body))
    return _conv_sc_cache[0](h, eterm, src, dst)


def _pack_rows(o):
    ilo = lax.bitcast_convert_type(o[:BE // 2, :], jnp.int32)
    ihi = lax.bitcast_convert_type(o[BE // 2:, :], jnp.int32)
    lo16 = lax.shift_right_logical(ilo + jnp.int32(32768), 16)
    hi16 = (ihi + jnp.int32(32768)) & jnp.int32(-65536)
    return lo16 | hi16


def _eterm_body(ea, w1, b1, wa, ba, wb, bb, wc, bc, o1, o2, o3):
    h = jnp.maximum(
        jnp.dot(ea[...], w1[...], preferred_element_type=jnp.float32) + b1[...],
        0.0).astype(jnp.bfloat16)
    wa_ = wa[...].astype(jnp.bfloat16)
    wb_ = wb[...].astype(jnp.bfloat16)
    wc_ = wc[...].astype(jnp.bfloat16)
    o1[...] = _pack_rows(
        jnp.dot(h, wa_, preferred_element_type=jnp.float32) + ba[...])
    o2[...] = _pack_rows(
        jnp.dot(h, wb_, preferred_element_type=jnp.float32) + bb[...])
    o3[...] = _pack_rows(
        jnp.dot(h, wc_, preferred_element_type=jnp.float32) + bc[...])


def _eterm_call(edge_attr, ep_W1, ep_b1, Wa, ba, Wb, bb, Wc, bc):
    full = lambda shape: pl.BlockSpec(shape, lambda i: (0, 0))
    out = jax.ShapeDtypeStruct((E // 2, H), jnp.int32)
    return pl.pallas_call(
        _eterm_body,
        grid=(E // BE,),
        in_specs=[
            pl.BlockSpec((BE, 16), lambda i: (i, 0)),
            full((16, H)), full((1, H)),
            full((H, H)), full((1, H)),
            full((H, H)), full((1, H)),
            full((H, H)), full((1, H)),
        ],
        out_specs=[pl.BlockSpec((BE // 2, H), lambda i: (i, 0))] * 3,
        out_shape=[out, out, out],
    )(edge_attr, ep_W1, ep_b1.reshape(1, H), Wa, ba.reshape(1, H),
      Wb, bb.reshape(1, H), Wc, bc.reshape(1, H))


def _mlp_body(x, a0, a1, w1, b1, w2, b2, o):
    z = x[...] + a0[...] + a1[...]
    y = jnp.maximum(
        jnp.dot(z, w1[...], preferred_element_type=jnp.float32) + b1[...], 0.0)
    o[...] = jnp.maximum(
        jnp.dot(y, w2[...], preferred_element_type=jnp.float32) + b2[...], 0.0)


def _mlp_call(x, agg2, W1, b1, W2, b2):
    full = lambda shape: pl.BlockSpec(shape, lambda i: (0, 0))
    nb = N // BN_
    return pl.pallas_call(
        _mlp_body,
        grid=(nb,),
        in_specs=[
            pl.BlockSpec((BN_, H), lambda i: (i, 0)),
            pl.BlockSpec((BN_, H), lambda i: (i, 0)),
            pl.BlockSpec((BN_, H), lambda i, nb=nb: (i + nb, 0)),
            full((H, H)), full((1, H)),
            full((H, H)), full((1, H)),
        ],
        out_specs=pl.BlockSpec((BN_, H), lambda i: (i, 0)),
        out_shape=jax.ShapeDtypeStruct((N, H), jnp.float32),
    )(x, agg2, agg2, W1, b1.reshape(1, H), W2, b2.reshape(1, H))


def _head_body(h1, h2, h3, bt, fw1, fb1, fw2, fb2, o, pooled):
    i = pl.program_id(0)

    @pl.when(i == 0)
    def _():
        pooled[...] = jnp.zeros((NG, 3 * H), jnp.float32)

    ids = bt[...].reshape(1, BN_)
    oh = (lax.broadcasted_iota(jnp.int32, (NG, BN_), 0) == ids
          ).astype(jnp.float32)
    pooled[:, 0:H] += jnp.dot(oh, h1[...], preferred_element_type=jnp.float32)
    pooled[:, H:2 * H] += jnp.dot(oh, h2[...],
                                  preferred_element_type=jnp.float32)
    pooled[:, 2 * H:3 * H] += jnp.dot(oh, h3[...],
                                      preferred_element_type=jnp.float32)

    @pl.when(i == pl.num_programs(0) - 1)
    def _():
        y = jnp.maximum(
            jnp.dot(pooled[...], fw1[...],
                    preferred_element_type=jnp.float32) + fb1[...], 0.0)
        o[...] = jnp.dot(y, fw2[...],
                         preferred_element_type=jnp.float32) + fb2[...]


def _head_call(h1, h2, h3, batch, fc1_W, fc1_b, fc2_W, fc2_b, out_dim):
    nb = N // BN_
    full = lambda shape: pl.BlockSpec(shape, lambda i: tuple(0 for _ in shape))
    nblk = pl.BlockSpec((BN_, H), lambda i: (i, 0))
    return pl.pallas_call(
        _head_body,
        grid=(nb,),
        in_specs=[
            nblk, nblk, nblk,
            pl.BlockSpec((1, 1, BN_), lambda i: (i, 0, 0)),
            full((3 * H, 3 * H)), full((1, 3 * H)),
            full((3 * H, out_dim)), full((1, out_dim)),
        ],
        out_specs=pl.BlockSpec((NG, out_dim), lambda i: (0, 0)),
        out_shape=jax.ShapeDtypeStruct((NG, out_dim), jnp.float32),
        scratch_shapes=[pltpu.VMEM((NG, 3 * H), jnp.float32)],
    )(h1, h2, h3, batch.reshape(nb, 1, BN_), fc1_W,
      fc1_b.reshape(1, 3 * H), fc2_W, fc2_b.reshape(1, out_dim))


def kernel(x, edge_index, edge_attr, batch, ep_W1, ep_b1, ep_W2, ep_b2,
           lin1_W, lin1_b, lin2_W, lin2_b, lin3_W, lin3_b,
           imn_W1, imn_b1, imn_g, imn_be, imn_m, imn_v, imn_W2, imn_b2,
           hmn_W1, hmn_b1, hmn_g, hmn_be, hmn_m, hmn_v, hmn_W2, hmn_b2,
           fc1_W, fc1_b, fc2_W, fc2_b):
    src = edge_index[0]
    dst = edge_index[1]
    out_dim = fc2_W.shape[1]

    Wa = ep_W2 @ lin1_W
    ba = ep_b2 @ lin1_W + lin1_b
    Wb = ep_W2 @ lin2_W
    bb = ep_b2 @ lin2_W + lin2_b
    Wc = ep_W2 @ lin3_W
    bc = ep_b2 @ lin3_W + lin3_b

    si = imn_g * jax.lax.rsqrt(imn_v + 1e-5)
    imn_W1f = imn_W1 * si[None, :]
    imn_b1f = imn_b1 * si + (imn_be - imn_m * si)
    sh = hmn_g * jax.lax.rsqrt(hmn_v + 1e-5)
    hmn_W1f = hmn_W1 * sh[None, :]
    hmn_b1f = hmn_b1 * sh + (hmn_be - hmn_m * sh)

    src_r = src.reshape(E // BE, 2, BE // 2).transpose(0, 2, 1) \
        .reshape(E // K, K)
    dst_r = dst.reshape(E // BE, 2, BE // 2).transpose(0, 2, 1) \
        .reshape(E // K, K)

    e1, e2, e3 = _eterm_call(edge_attr, ep_W1, ep_b1, Wa, ba, Wb, bb, Wc, bc)

    a1 = _conv_sc(x, e1, src_r, dst_r)
    h1 = _mlp_call(x, a1, imn_W1f, imn_b1f, imn_W2, imn_b2)
    a2 = _conv_sc(h1, e2, src_r, dst_r)
    h2 = _mlp_call(h1, a2, hmn_W1f, hmn_b1f, hmn_W2, hmn_b2)
    a3 = _conv_sc(h2, e3, src_r, dst_r)
    h3 = _mlp_call(h2, a3, hmn_W1f, hmn_b1f, hmn_W2, hmn_b2)

    return _head_call(h1, h2, h3, batch, fc1_W, fc1_b, fc2_W, fc2_b, out_dim)


# speedup vs baseline: 1.4235x; 1.4235x over previous
import functools

import jax
import jax.numpy as jnp
import numpy as np
from jax import lax
from jax.experimental import pallas as pl
from jax.experimental.pallas import tpu as pltpu
from jax.experimental.pallas import tpu_sc as plsc

N = 10000
E = 320000
H = 128
NG = 64

NC = 2
NS = 16
K = 40
EPW = E // (NC * NS)
NCHUNK = EPW // K
NSLOT = 4
LDA = 2
IXA = 3
ZC = 40
NZCH = N // ZC

BE = 512
BN_ = 1000


def _conv_sc_body(h_hbm, eterm_hbm, src_hbm, dst_hbm, out_hbm, *rest):
    srcv = list(rest[0:NSLOT])
    dstv = list(rest[NSLOT:2 * NSLOT])
    ebufs = list(rest[2 * NSLOT:3 * NSLOT])
    hbufs = list(rest[3 * NSLOT:4 * NSLOT])
    aggr_sh = rest[4 * NSLOT]
    isems = list(rest[4 * NSLOT + 1:5 * NSLOT + 1])
    lsems = list(rest[5 * NSLOT + 1:6 * NSLOT + 1])
    ssems = list(rest[6 * NSLOT + 1:7 * NSLOT + 1])
    c = lax.axis_index("c")
    s = lax.axis_index("s")
    wid = s * NC + c
    zbuf = hbufs[0]

    def wait_rows(dst_ref, sem):
        pltpu.make_async_copy(h_hbm.at[pl.ds(0, K)], dst_ref, sem).wait()

    def wait_eterm(dst_ref, sem):
        pltpu.make_async_copy(eterm_hbm.at[pl.ds(0, K)], dst_ref,
                              sem).wait()

    def wait_idx(dst_ref, sem):
        pltpu.make_async_copy(src_hbm.at[0], dst_ref, sem).wait()

    def zrow(r, carry):
        for v in range(H // 16):
            zbuf[r, pl.ds(v * 16, 16)] = jnp.zeros((16,), jnp.float32)
        return carry
    lax.fori_loop(0, ZC, zrow, 0)

    def zchunk(t, carry):
        j = s + t * NS

        @pl.when(j < NZCH)
        def _():
            pltpu.sync_copy(zbuf.at[pl.ds(0, ZC)],
                            aggr_sh.at[pl.ds(j * ZC, ZC)])
        return carry
    lax.fori_loop(0, (NZCH + NS - 1) // NS, zchunk, 0)
    plsc.subcore_barrier()

    ebase = wid * EPW
    cbase = wid * NCHUNK

    def issue_src_idx(j, sl):
        pltpu.async_copy(src_hbm.at[cbase + j], srcv[sl], isems[sl])

    def issue_loads(j, sl):
        pltpu.async_copy(dst_hbm.at[cbase + j], dstv[sl], lsems[sl])
        pltpu.async_copy(eterm_hbm.at[pl.ds(ebase + j * K, K)],
                         ebufs[sl], lsems[sl])
        pltpu.async_copy(h_hbm.at[srcv[sl]], hbufs[sl], lsems[sl])

    def relu_add(sl):
        def row(r, rc):
            for g in range(H // 16):
                sg = pl.ds(16 * g, 16)
                hbufs[sl][r, sg] = jnp.maximum(
                    hbufs[sl][r, sg] + ebufs[sl][r, sg], 0.0)
            return rc
        lax.fori_loop(0, K, row, 0)

    def scatter(j, sl):
        pltpu.async_copy(hbufs[sl], aggr_sh.at[dstv[sl]], ssems[sl],
                         add=True)

    for j0 in range(LDA):
        pltpu.sync_copy(src_hbm.at[cbase + j0], srcv[j0])
    for j0 in range(LDA, IXA):
        issue_src_idx(j0, j0 % NSLOT)
    for j0 in range(LDA):
        issue_loads(j0, j0)

    def grp(t, carry):
        for u in range(NSLOT):
            j = NSLOT * t + u
            sp = (u + IXA) % NSLOT
            sn = (u + LDA) % NSLOT

            @pl.when(j + IXA < NCHUNK)
            def _():
                issue_src_idx(j + IXA, sp)

            @pl.when((j >= NSLOT - LDA) & (j + LDA < NCHUNK))
            def _():
                wait_rows(hbufs[sn], ssems[sn])

            @pl.when(j + LDA < NCHUNK)
            def _():
                wait_idx(srcv[sn], isems[sn])
                issue_loads(j + LDA, sn)

            wait_idx(dstv[u], lsems[u])
            wait_eterm(ebufs[u], lsems[u])
            wait_rows(hbufs[u], lsems[u])
            relu_add(u)
            scatter(j, u)
        return carry
    lax.fori_loop(0, NCHUNK // NSLOT, grp, 0)
    for j in range((NCHUNK // NSLOT) * NSLOT, NCHUNK):
        u = j % NSLOT
        wait_idx(dstv[u], lsems[u])
        wait_eterm(ebufs[u], lsems[u])
        wait_rows(hbufs[u], lsems[u])
        relu_add(u)
        scatter(j, u)
    for j in range(NCHUNK - NSLOT, NCHUNK):
        wait_rows(hbufs[j % NSLOT], ssems[j % NSLOT])
    plsc.subcore_barrier()

    def exp(t, carry):
        j = s + t * NS

        @pl.when(j < NZCH)
        def _():
            pltpu.sync_copy(aggr_sh.at[pl.ds(j * ZC, ZC)],
                            zbuf.at[pl.ds(0, ZC)])
            pltpu.sync_copy(zbuf.at[pl.ds(0, ZC)],
                            out_hbm.at[pl.ds(c * N + j * ZC, ZC)])
        return carry
    lax.fori_loop(0, (NZCH + NS - 1) // NS, exp, 0)


_conv_sc_cache = []


def _conv_sc(h, eterm, src, dst):
    if not _conv_sc_cache:
        _conv_sc_cache.append(functools.partial(
            pl.kernel,
            mesh=plsc.VectorSubcoreMesh(core_axis_name="c",
                                        subcore_axis_name="s"),
            out_type=jax.ShapeDtypeStruct((NC * N, H), jnp.float32),
            scratch_types=(
                [pltpu.VMEM((K,), jnp.int32)] * NSLOT
                + [pltpu.VMEM((K,), jnp.int32)] * NSLOT
                + [pltpu.VMEM((K, H), jnp.float32)] * NSLOT
                + [pltpu.VMEM((K, H), jnp.float32)] * NSLOT
                + [pltpu.VMEM_SHARED((N, H), jnp.float32)]
                + [pltpu.SemaphoreType.DMA] * (3 * NSLOT)
            ),
        )(_conv_sc_body))
    return _conv_sc_cache[0](h, eterm, src, dst)


def _eterm_body(ea, w1, b1, wa, ba, wb, bb, wc, bc, o1, o2, o3):
    h = jnp.maximum(
        jnp.dot(ea[...], w1[...], preferred_element_type=jnp.float32) + b1[...],
        0.0).astype(jnp.bfloat16)
    wa_ = wa[...].astype(jnp.bfloat16)
    wb_ = wb[...].astype(jnp.bfloat16)
    wc_ = wc[...].astype(jnp.bfloat16)
    o1[...] = jnp.dot(h, wa_, preferred_element_type=jnp.float32) + ba[...]
    o2[...] = jnp.dot(h, wb_, preferred_element_type=jnp.float32) + bb[...]
    o3[...] = jnp.dot(h, wc_, preferred_element_type=jnp.float32) + bc[...]


def _eterm_call(edge_attr, ep_W1, ep_b1, Wa, ba, Wb, bb, Wc, bc):
    full = lambda shape: pl.BlockSpec(shape, lambda i: (0, 0))
    out = jax.ShapeDtypeStruct((E, H), jnp.float32)
    return pl.pallas_call(
        _eterm_body,
        grid=(E // BE,),
        in_specs=[
            pl.BlockSpec((BE, 16), lambda i: (i, 0)),
            full((16, H)), full((1, H)),
            full((H, H)), full((1, H)),
            full((H, H)), full((1, H)),
            full((H, H)), full((1, H)),
        ],
        out_specs=[pl.BlockSpec((BE, H), lambda i: (i, 0))] * 3,
        out_shape=[out, out, out],
    )(edge_attr, ep_W1, ep_b1.reshape(1, H), Wa, ba.reshape(1, H),
      Wb, bb.reshape(1, H), Wc, bc.reshape(1, H))


def _mlp_body(x, a0, a1, w1, b1, w2, b2, o):
    z = x[...] + a0[...] + a1[...]
    y = jnp.maximum(
        jnp.dot(z, w1[...], preferred_element_type=jnp.float32) + b1[...], 0.0)
    o[...] = jnp.maximum(
        jnp.dot(y, w2[...], preferred_element_type=jnp.float32) + b2[...], 0.0)


def _mlp_call(x, agg2, W1, b1, W2, b2):
    full = lambda shape: pl.BlockSpec(shape, lambda i: (0, 0))
    nb = N // BN_
    return pl.pallas_call(
        _mlp_body,
        grid=(nb,),
        in_specs=[
            pl.BlockSpec((BN_, H), lambda i: (i, 0)),
            pl.BlockSpec((BN_, H), lambda i: (i, 0)),
            pl.BlockSpec((BN_, H), lambda i, nb=nb: (i + nb, 0)),
            full((H, H)), full((1, H)),
            full((H, H)), full((1, H)),
        ],
        out_specs=pl.BlockSpec((BN_, H), lambda i: (i, 0)),
        out_shape=jax.ShapeDtypeStruct((N, H), jnp.float32),
    )(x, agg2, agg2, W1, b1.reshape(1, H), W2, b2.reshape(1, H))


def _head_body(h1, h2, h3, bt, fw1, fb1, fw2, fb2, o, pooled):
    i = pl.program_id(0)

    @pl.when(i == 0)
    def _():
        pooled[...] = jnp.zeros((NG, 3 * H), jnp.float32)

    ids = bt[...].reshape(1, BN_)
    oh = (lax.broadcasted_iota(jnp.int32, (NG, BN_), 0) == ids
          ).astype(jnp.float32)
    pooled[:, 0:H] += jnp.dot(oh, h1[...], preferred_element_type=jnp.float32)
    pooled[:, H:2 * H] += jnp.dot(oh, h2[...],
                                  preferred_element_type=jnp.float32)
    pooled[:, 2 * H:3 * H] += jnp.dot(oh, h3[...],
                                      preferred_element_type=jnp.float32)

    @pl.when(i == pl.num_programs(0) - 1)
    def _():
        y = jnp.maximum(
            jnp.dot(pooled[...], fw1[...],
                    preferred_element_type=jnp.float32) + fb1[...], 0.0)
        o[...] = jnp.dot(y, fw2[...],
                         preferred_element_type=jnp.float32) + fb2[...]


def _head_call(h1, h2, h3, batch, fc1_W, fc1_b, fc2_W, fc2_b, out_dim):
    nb = N // BN_
    full = lambda shape: pl.BlockSpec(shape, lambda i: tuple(0 for _ in shape))
    nblk = pl.BlockSpec((BN_, H), lambda i: (i, 0))
    return pl.pallas_call(
        _head_body,
        grid=(nb,),
        in_specs=[
            nblk, nblk, nblk,
            pl.BlockSpec((1, 1, BN_), lambda i: (i, 0, 0)),
            full((3 * H, 3 * H)), full((1, 3 * H)),
            full((3 * H, out_dim)), full((1, out_dim)),
        ],
        out_specs=pl.BlockSpec((NG, out_dim), lambda i: (0, 0)),
        out_shape=jax.ShapeDtypeStruct((NG, out_dim), jnp.float32),
        scratch_shapes=[pltpu.VMEM((NG, 3 * H), jnp.float32)],
    )(h1, h2, h3, batch.reshape(nb, 1, BN_), fc1_W,
      fc1_b.reshape(1, 3 * H), fc2_W, fc2_b.reshape(1, out_dim))


def kernel(x, edge_index, edge_attr, batch, ep_W1, ep_b1, ep_W2, ep_b2,
           lin1_W, lin1_b, lin2_W, lin2_b, lin3_W, lin3_b,
           imn_W1, imn_b1, imn_g, imn_be, imn_m, imn_v, imn_W2, imn_b2,
           hmn_W1, hmn_b1, hmn_g, hmn_be, hmn_m, hmn_v, hmn_W2, hmn_b2,
           fc1_W, fc1_b, fc2_W, fc2_b):
    src = edge_index[0]
    dst = edge_index[1]
    out_dim = fc2_W.shape[1]

    Wa = ep_W2 @ lin1_W
    ba = ep_b2 @ lin1_W + lin1_b
    Wb = ep_W2 @ lin2_W
    bb = ep_b2 @ lin2_W + lin2_b
    Wc = ep_W2 @ lin3_W
    bc = ep_b2 @ lin3_W + lin3_b

    si = imn_g * jax.lax.rsqrt(imn_v + 1e-5)
    imn_W1f = imn_W1 * si[None, :]
    imn_b1f = imn_b1 * si + (imn_be - imn_m * si)
    sh = hmn_g * jax.lax.rsqrt(hmn_v + 1e-5)
    hmn_W1f = hmn_W1 * sh[None, :]
    hmn_b1f = hmn_b1 * sh + (hmn_be - hmn_m * sh)

    src_r = src.reshape(E // K, K)
    dst_r = dst.reshape(E // K, K)

    e1, e2, e3 = _eterm_call(edge_attr, ep_W1, ep_b1, Wa, ba, Wb, bb, Wc, bc)

    a1 = _conv_sc(x, e1, src_r, dst_r)
    h1 = _mlp_call(x, a1, imn_W1f, imn_b1f, imn_W2, imn_b2)
    a2 = _conv_sc(h1, e2, src_r, dst_r)
    h2 = _mlp_call(h1, a2, hmn_W1f, hmn_b1f, hmn_W2, hmn_b2)
    a3 = _conv_sc(h2, e3, src_r, dst_r)
    h3 = _mlp_call(h2, a3, hmn_W1f, hmn_b1f, hmn_W2, hmn_b2)

    return _head_call(h1, h2, h3, batch, fc1_W, fc1_b, fc2_W, fc2_b, out_dim)
